# speedup vs baseline: 9.3225x; 9.3225x over previous
import functools

import jax
import jax.numpy as jnp
from jax.experimental import pallas as pl
from jax.experimental.pallas import tpu as pltpu

D_MICRO = 11
D_MODEL = 256
KTOP = 32
N_CELLS = 4096
NB = 256

_F32 = jnp.float32


def _prep_kernel(macro_ref, Wk_ref, Wv_ref, Wo_ref, kfull_ref, wvo_ref):
    b = pl.program_id(0)
    kfull_ref[0] = jnp.dot(macro_ref[0], Wk_ref[...],
                           preferred_element_type=_F32)

    @pl.when(b == 0)
    def _():
        wvo_ref[...] = jnp.dot(Wv_ref[...], Wo_ref[...],
                               preferred_element_type=_F32)


def _main_kernel(micro_ref, macro_ref, kfull_ref, Wp_ref, bp_ref, Wq_ref,
                 wvo_ref, bo_ref, xc_ref, aw_ref, ti_ref, s_scratch):
    scale = D_MODEL ** (-0.5)
    x = jnp.dot(micro_ref[0], Wp_ref[...],
                preferred_element_type=_F32) + bp_ref[...]
    q = jnp.dot(x, Wq_ref[...], preferred_element_type=_F32)
    s = jax.lax.dot_general(q, kfull_ref[0],
                            (((1,), (1,)), ((), ())),
                            preferred_element_type=_F32) * scale
    s_scratch[...] = s

    iota = jax.lax.broadcasted_iota(jnp.int32, (NB, N_CELLS), 1)
    lane = jax.lax.broadcasted_iota(jnp.int32, (NB, KTOP), 1)
    neg_inf = _F32(-jnp.inf)

    def body(j, carry):
        vals, idxs = carry
        s_c = s_scratch[...]
        m = jnp.max(s_c, axis=1, keepdims=True)
        is_m = s_c == m
        idx = jnp.min(jnp.where(is_m, iota, N_CELLS), axis=1, keepdims=True)
        s_scratch[...] = jnp.where(iota == idx, neg_inf, s_c)
        vals = jnp.where(lane == j, m, vals)
        idxs = jnp.where(lane == j, idx, idxs)
        return vals, idxs

    vals0 = jnp.zeros((NB, KTOP), _F32)
    idxs0 = jnp.zeros((NB, KTOP), jnp.int32)
    vals, idxs = jax.lax.fori_loop(0, KTOP, body, (vals0, idxs0))

    mx = vals[:, :1]
    ew = jnp.exp(vals - mx)
    z = jnp.sum(ew, axis=1, keepdims=True)
    aw_ref[0] = ew / z
    ti_ref[0] = idxs

    tau = vals[:, KTOP - 1:KTOP]
    p = jnp.where(s >= tau, jnp.exp(s - mx), _F32(0.0)) / z
    ctx = jnp.dot(p, macro_ref[0], preferred_element_type=_F32)
    xc_ref[0] = x + jnp.dot(ctx, wvo_ref[...],
                            preferred_element_type=_F32) + bo_ref[...]


@jax.jit
def kernel(micro_tokens, macro_output, Wp, bp, Wq, Wk, Wv, Wo, bo):
    B, N, _ = micro_tokens.shape
    kfull, wvo = pl.pallas_call(
        _prep_kernel,
        grid=(B,),
        in_specs=[
            pl.BlockSpec((1, N_CELLS, D_MODEL), lambda b: (b, 0, 0)),
            pl.BlockSpec((D_MODEL, D_MODEL), lambda b: (0, 0)),
            pl.BlockSpec((D_MODEL, D_MODEL), lambda b: (0, 0)),
            pl.BlockSpec((D_MODEL, D_MODEL), lambda b: (0, 0)),
        ],
        out_specs=[
            pl.BlockSpec((1, N_CELLS, D_MODEL), lambda b: (b, 0, 0)),
            pl.BlockSpec((D_MODEL, D_MODEL), lambda b: (0, 0)),
        ],
        out_shape=[
            jax.ShapeDtypeStruct((B, N_CELLS, D_MODEL), _F32),
            jax.ShapeDtypeStruct((D_MODEL, D_MODEL), _F32),
        ],
    )(macro_output, Wk, Wv, Wo)

    bp2 = bp.reshape(1, D_MODEL)
    bo2 = bo.reshape(1, D_MODEL)
    grid = (B, N // NB)
    xc, aw, ti = pl.pallas_call(
        _main_kernel,
        grid=grid,
        in_specs=[
            pl.BlockSpec((1, NB, D_MICRO), lambda b, n: (b, n, 0)),
            pl.BlockSpec((1, N_CELLS, D_MODEL), lambda b, n: (b, 0, 0)),
            pl.BlockSpec((1, N_CELLS, D_MODEL), lambda b, n: (b, 0, 0)),
            pl.BlockSpec((D_MICRO, D_MODEL), lambda b, n: (0, 0)),
            pl.BlockSpec((1, D_MODEL), lambda b, n: (0, 0)),
            pl.BlockSpec((D_MODEL, D_MODEL), lambda b, n: (0, 0)),
            pl.BlockSpec((D_MODEL, D_MODEL), lambda b, n: (0, 0)),
            pl.BlockSpec((1, D_MODEL), lambda b, n: (0, 0)),
        ],
        out_specs=[
            pl.BlockSpec((1, NB, D_MODEL), lambda b, n: (b, n, 0)),
            pl.BlockSpec((1, NB, KTOP), lambda b, n: (b, n, 0)),
            pl.BlockSpec((1, NB, KTOP), lambda b, n: (b, n, 0)),
        ],
        out_shape=[
            jax.ShapeDtypeStruct((B, N, D_MODEL), _F32),
            jax.ShapeDtypeStruct((B, N, KTOP), _F32),
            jax.ShapeDtypeStruct((B, N, KTOP), jnp.int32),
        ],
        scratch_shapes=[pltpu.VMEM((NB, N_CELLS), _F32)],
        compiler_params=pltpu.CompilerParams(
            dimension_semantics=("parallel", "arbitrary"),
        ),
    )(micro_tokens, macro_output, kfull, Wp, bp2, Wq, wvo, bo2)
    return (xc, aw, ti)


# speedup vs baseline: 13.0534x; 1.4002x over previous
import functools

import jax
import jax.numpy as jnp
from jax import lax
from jax.experimental import pallas as pl
from jax.experimental.pallas import tpu as pltpu
from jax.experimental.pallas import tpu_sc as plsc

D_MICRO = 11
D_MODEL = 256
KTOP = 32
N_CELLS = 4096
NB = 256
NBLK = 2048 // NB

NWORK = 32
ROWS = 8 * 2048
RPW = ROWS // NWORK
CHUNK = 64

_F32 = jnp.float32
_I32 = jnp.int32


def _prep_kernel(macro_ref, Wk_ref, Wv_ref, Wo_ref, kfull_ref, wvo_ref):
    b = pl.program_id(0)
    kfull_ref[0] = jnp.dot(macro_ref[0], Wk_ref[...],
                           preferred_element_type=_F32)

    @pl.when(b == 0)
    def _():
        wvo_ref[...] = jnp.dot(Wv_ref[...], Wo_ref[...],
                               preferred_element_type=_F32)


def _score_kernel(micro_ref, kfull_ref, Wp_ref, bp_ref, Wq_ref,
                  s_ref, x_ref, tau_ref):
    scale = D_MODEL ** (-0.5)
    x = jnp.dot(micro_ref[0], Wp_ref[...],
                preferred_element_type=_F32) + bp_ref[...]
    q = jnp.dot(x, Wq_ref[...], preferred_element_type=_F32)
    s = lax.dot_general(q, kfull_ref[0], (((1,), (1,)), ((), ())),
                        preferred_element_type=_F32) * scale
    s_ref[0] = s
    x_ref[0] = x
    cm = jnp.max(s.reshape(NB, KTOP, N_CELLS // KTOP), axis=2)
    tau_ref[0, 0] = jnp.min(cm, axis=1)


def _sc_topk_kernel(scores, tau, aw_out, ti_out, st_out,
                    rowbuf, tauv, cv, ci, awb, tib, stb, sem0, sem1):
    wid = lax.axis_index("s") * 2 + lax.axis_index("c")
    base = pl.multiple_of(wid * RPW, RPW)
    pltpu.sync_copy(tau.at[pl.ds(base, RPW)], tauv.at[pl.ds(0, RPW)])
    pltpu.async_copy(scores.at[pl.ds(base * N_CELLS, N_CELLS)],
                     rowbuf.at[pl.ds(0, N_CELLS)], sem0)
    pltpu.async_copy(scores.at[pl.ds((base + 1) * N_CELLS, N_CELLS)],
                     rowbuf.at[pl.ds(N_CELLS, N_CELLS)], sem1)
    iota16 = lax.iota(_I32, 16)
    neg = _F32(-jnp.inf)
    neg16 = jnp.full((16,), neg, _F32)
    zero16 = jnp.zeros((16,), _I32)

    def process_row(r, slot, sem):
        sbase = slot * N_CELLS
        pltpu.make_async_copy(
            scores.at[pl.ds((base + r) * N_CELLS, N_CELLS)],
            rowbuf.at[pl.ds(sbase, N_CELLS)], sem).wait()
        tau_s = tauv[pl.ds(r, 16)][0]

        def scan8(j8, off):
            for u in range(8):
                j = j8 * 8 + u
                v = rowbuf[pl.ds(sbase + j * 16, 16)]
                m = v >= tau_s
                plsc.store_compressed(cv.at[pl.ds(off, 16)], v, mask=m)
                plsc.store_compressed(ci.at[pl.ds(off, 16)],
                                      iota16 + j * 16, mask=m)
                off = off + plsc.all_reduce_population_count(m)[0]
            return off

        off = lax.fori_loop(0, N_CELLS // 128, scan8, _I32(0))
        cv[pl.ds(off, 16)] = neg16
        ci[pl.ds(off, 16)] = zero16
        nv = lax.shift_right_logical(off + 15, 4)

        def merge(c, carry):
            r0v, r0i, r1v, r1i = carry
            kv = cv[pl.ds(c * 16, 16)]
            ki = ci[pl.ds(c * 16, 16)]
            kv, ki = plsc.sort_key_val(kv, ki, descending=True)
            rkv = lax.rev(kv, (0,))
            rki = lax.rev(ki, (0,))
            m = r1v >= rkv
            sv = jnp.where(m, r1v, rkv)
            si = jnp.where(m, r1i, rki)
            sv, si = plsc.sort_key_val(sv, si, descending=True)
            rsv = lax.rev(sv, (0,))
            rsi = lax.rev(si, (0,))
            m2 = r0v >= rsv
            hv = jnp.where(m2, r0v, rsv)
            hi = jnp.where(m2, r0i, rsi)
            lv = jnp.where(m2, rsv, r0v)
            li = jnp.where(m2, rsi, r0i)
            hv, hi = plsc.sort_key_val(hv, hi, descending=True)
            lv, li = plsc.sort_key_val(lv, li, descending=True)
            return hv, hi, lv, li

        r0v, r0i, r1v, r1i = lax.fori_loop(
            0, nv, merge, (neg16, zero16, neg16, zero16))

        mx = jnp.max(r0v)
        e0 = jnp.exp(r0v - mx)
        e1 = jnp.exp(r1v - mx)
        z = jnp.sum(e0) + jnp.sum(e1)
        inv_vec = jnp.full((16,), _F32(1.0), _F32) / z
        rc = lax.rem(r, CHUNK)
        awb[pl.ds(rc * KTOP, 16)] = e0 * inv_vec
        awb[pl.ds(rc * KTOP + 16, 16)] = e1 * inv_vec
        tib[pl.ds(rc * KTOP, 16)] = r0i
        tib[pl.ds(rc * KTOP + 16, 16)] = r1i
        tau32 = jnp.min(r1v)
        stvec = jnp.where(iota16 == 0, mx,
                          jnp.where(iota16 == 1, inv_vec,
                                    jnp.full((16,), tau32, _F32)))
        stb[pl.ds(rc * 16, 16)] = stvec

        @pl.when(rc == CHUNK - 1)
        def _():
            rb = base + r - (CHUNK - 1)
            pltpu.sync_copy(awb, aw_out.at[pl.ds(rb * KTOP, CHUNK * KTOP)])
            pltpu.sync_copy(tib, ti_out.at[pl.ds(rb * KTOP, CHUNK * KTOP)])
            pltpu.sync_copy(stb, st_out.at[pl.ds(rb * 16, CHUNK * 16)])

    def pair_body(i, _):
        r0 = i * 2
        process_row(r0, 0, sem0)

        @pl.when(r0 + 2 < RPW)
        def _():
            pltpu.async_copy(
                scores.at[pl.ds((base + r0 + 2) * N_CELLS, N_CELLS)],
                rowbuf.at[pl.ds(0, N_CELLS)], sem0)

        process_row(r0 + 1, 1, sem1)

        @pl.when(r0 + 3 < RPW)
        def _():
            pltpu.async_copy(
                scores.at[pl.ds((base + r0 + 3) * N_CELLS, N_CELLS)],
                rowbuf.at[pl.ds(N_CELLS, N_CELLS)], sem1)

        return 0

    lax.fori_loop(0, RPW // 2, pair_body, 0)


def _combine_kernel(s_ref, x_ref, st_ref, macro_ref, wvo_ref, bo_ref,
                    xc_ref):
    s = s_ref[...]
    mx = st_ref[:, 0:1]
    inv = st_ref[:, 1:2]
    tau32 = st_ref[:, 2:3]
    p = jnp.where(s >= tau32, jnp.exp(s - mx), _F32(0.0)) * inv
    ctx = jnp.dot(p, macro_ref[0], preferred_element_type=_F32)
    xc_ref[...] = x_ref[...] + jnp.dot(ctx, wvo_ref[...],
                                       preferred_element_type=_F32) \
        + bo_ref[...]


@jax.jit
def kernel(micro_tokens, macro_output, Wp, bp, Wq, Wk, Wv, Wo, bo):
    B, N, _ = micro_tokens.shape
    kfull, wvo = pl.pallas_call(
        _prep_kernel,
        grid=(B,),
        in_specs=[
            pl.BlockSpec((1, N_CELLS, D_MODEL), lambda b: (b, 0, 0)),
            pl.BlockSpec((D_MODEL, D_MODEL), lambda b: (0, 0)),
            pl.BlockSpec((D_MODEL, D_MODEL), lambda b: (0, 0)),
            pl.BlockSpec((D_MODEL, D_MODEL), lambda b: (0, 0)),
        ],
        out_specs=[
            pl.BlockSpec((1, N_CELLS, D_MODEL), lambda b: (b, 0, 0)),
            pl.BlockSpec((D_MODEL, D_MODEL), lambda b: (0, 0)),
        ],
        out_shape=[
            jax.ShapeDtypeStruct((B, N_CELLS, D_MODEL), _F32),
            jax.ShapeDtypeStruct((D_MODEL, D_MODEL), _F32),
        ],
    )(macro_output, Wk, Wv, Wo)

    bp2 = bp.reshape(1, D_MODEL)
    bo2 = bo.reshape(1, D_MODEL)
    scores, x_micro, tau = pl.pallas_call(
        _score_kernel,
        grid=(B, NBLK),
        in_specs=[
            pl.BlockSpec((1, NB, D_MICRO), lambda b, n: (b, n, 0)),
            pl.BlockSpec((1, N_CELLS, D_MODEL), lambda b, n: (b, 0, 0)),
            pl.BlockSpec((D_MICRO, D_MODEL), lambda b, n: (0, 0)),
            pl.BlockSpec((1, D_MODEL), lambda b, n: (0, 0)),
            pl.BlockSpec((D_MODEL, D_MODEL), lambda b, n: (0, 0)),
        ],
        out_specs=[
            pl.BlockSpec((1, NB, N_CELLS), lambda b, n: (b, n, 0)),
            pl.BlockSpec((1, NB, D_MODEL), lambda b, n: (b, n, 0)),
            pl.BlockSpec((1, 1, NB), lambda b, n: (b * NBLK + n, 0, 0)),
        ],
        out_shape=[
            jax.ShapeDtypeStruct((B, N, N_CELLS), _F32),
            jax.ShapeDtypeStruct((B, N, D_MODEL), _F32),
            jax.ShapeDtypeStruct((B * NBLK, 1, NB), _F32),
        ],
        compiler_params=pltpu.CompilerParams(
            dimension_semantics=("parallel", "arbitrary"),
        ),
    )(micro_tokens, kfull, Wp, bp2, Wq)

    scores_flat = scores.reshape(ROWS, N_CELLS)
    scores_1d = scores.reshape(ROWS * N_CELLS)
    tau_flat = tau.reshape(ROWS)

    mesh = plsc.VectorSubcoreMesh(core_axis_name="c", subcore_axis_name="s",
                                  num_cores=2, num_subcores=16)
    aw_flat, ti_flat, stats = pl.kernel(
        _sc_topk_kernel,
        out_type=[
            jax.ShapeDtypeStruct((ROWS * KTOP,), _F32),
            jax.ShapeDtypeStruct((ROWS * KTOP,), _I32),
            jax.ShapeDtypeStruct((ROWS * 16,), _F32),
        ],
        mesh=mesh,
        compiler_params=pltpu.CompilerParams(needs_layout_passes=False),
        scratch_types=[
            pltpu.VMEM((2 * N_CELLS,), _F32),
            pltpu.VMEM((RPW + 16,), _F32),
            pltpu.VMEM((N_CELLS + 16, ), _F32),
            pltpu.VMEM((N_CELLS + 16, ), _I32),
            pltpu.VMEM((CHUNK * KTOP,), _F32),
            pltpu.VMEM((CHUNK * KTOP,), _I32),
            pltpu.VMEM((CHUNK * 16,), _F32),
            pltpu.SemaphoreType.DMA,
            pltpu.SemaphoreType.DMA,
        ],
    )(scores_1d, tau_flat)
    stats = stats.reshape(ROWS, 16)

    xc_flat = pl.pallas_call(
        _combine_kernel,
        grid=(B * NBLK,),
        in_specs=[
            pl.BlockSpec((NB, N_CELLS), lambda i: (i, 0)),
            pl.BlockSpec((NB, D_MODEL), lambda i: (i, 0)),
            pl.BlockSpec((NB, 16), lambda i: (i, 0)),
            pl.BlockSpec((1, N_CELLS, D_MODEL), lambda i: (i // NBLK, 0, 0)),
            pl.BlockSpec((D_MODEL, D_MODEL), lambda i: (0, 0)),
            pl.BlockSpec((1, D_MODEL), lambda i: (0, 0)),
        ],
        out_specs=pl.BlockSpec((NB, D_MODEL), lambda i: (i, 0)),
        out_shape=jax.ShapeDtypeStruct((ROWS, D_MODEL), _F32),
    )(scores_flat, x_micro.reshape(ROWS, D_MODEL), stats, macro_output,
      wvo, bo2)

    return (xc_flat.reshape(B, N, D_MODEL),
            aw_flat.reshape(B, N, KTOP),
            ti_flat.reshape(B, N, KTOP))
